# baseline (speedup 1.0000x reference)
import functools

import jax
import jax.numpy as jnp
from jax import lax
from jax.experimental import pallas as pl
from jax.experimental.pallas import tpu as pltpu
from jax.experimental.pallas import tpu_sc as plsc

_CB = 80
_NRINGS = 4
_NDIRS = 8
_NPOS = _NRINGS * _NDIRS


def _sc_gather(table, idx_flat, nv):
    nch = table.shape[1]
    gw = _NDIRS * nch
    info = plsc.get_sparse_core_info()
    nw = info.num_cores * info.num_subcores
    n_chunks = nv // _CB
    base_per_w = n_chunks // nw
    extra = n_chunks - base_per_w * nw
    mesh = plsc.VectorSubcoreMesh(core_axis_name="c", subcore_axis_name="s")

    @functools.partial(
        pl.kernel,
        mesh=mesh,
        compiler_params=pltpu.CompilerParams(
            use_tc_tiling_on_sc=False, needs_layout_passes=False),
        out_type=[jax.ShapeDtypeStruct((nv, gw), table.dtype)
                  for _ in range(_NRINGS)],
        scratch_types=[
            pltpu.VMEM((_CB, _NRINGS, _NDIRS), jnp.int32),
            pltpu.VMEM((2, _NPOS, _CB), jnp.int32),
            pltpu.VMEM((2, _NRINGS, _NDIRS, _CB, nch), table.dtype),
            pltpu.SemaphoreType.DMA((2, _NRINGS)),
            pltpu.SemaphoreType.DMA((2, _NRINGS)),
        ],
    )
    def gather_kernel(table_hbm, idx_hbm, o0, o1, o2, o3,
                      idxv, idxp, rows_v, gsem, wsem):
        outs = [o0, o1, o2, o3]
        wid = lax.axis_index("s") * info.num_cores + lax.axis_index("c")
        start_w = wid * base_per_w + lax.min(wid, extra)
        n_w = base_per_w + jnp.where(wid < extra, 1, 0)
        lane = lax.iota(jnp.int32, 16)
        ones = jnp.full((16,), 1, jnp.int32)

        def gathers_start(b, chunk):
            for r in range(_NRINGS):
                for dd in range(_NDIRS):
                    pltpu.async_copy(
                        table_hbm.at[idxp.at[b, r * _NDIRS + dd]],
                        rows_v.at[b, r, dd], gsem.at[b, r])

        def gathers_wait(b):
            for r in range(_NRINGS):
                for dd in range(_NDIRS):
                    pltpu.make_async_copy(
                        table_hbm.at[idxp.at[b, r * _NDIRS + dd]],
                        rows_v.at[b, r, dd], gsem.at[b, r]).wait()

        def wb_start(b, v0):
            for r in range(_NRINGS):
                for dd in range(_NDIRS):
                    pltpu.async_copy(
                        rows_v.at[b, r, dd],
                        outs[r].at[pl.ds(v0, _CB), pl.ds(dd * nch, nch)],
                        wsem.at[b, r])

        def wb_wait(b, v0):
            for r in range(_NRINGS):
                for dd in range(_NDIRS):
                    pltpu.make_async_copy(
                        rows_v.at[b, r, dd],
                        outs[r].at[pl.ds(v0, _CB), pl.ds(dd * nch, nch)],
                        wsem.at[b, r]).wait()

        def body(j, carry):
            b = lax.rem(j, 2)
            pb = 1 - b
            chunk = start_w + j
            v0 = chunk * _CB
            pltpu.sync_copy(idx_hbm.at[0, pl.ds(v0, _CB)], idxv)
            for r in range(_NRINGS):
                for dd in range(_NDIRS):
                    p = r * _NDIRS + dd
                    for k in range(_CB // 16):
                        vals = plsc.load_gather(
                            idxv, [lane + 16 * k, ones * r, ones * dd])
                        idxp[b, p, pl.ds(16 * k, 16)] = vals

            @pl.when(j > 1)
            def _():
                wb_wait(b, v0 - 2 * _CB)

            gathers_start(b, chunk)

            @pl.when(j > 0)
            def _():
                gathers_wait(pb)
                wb_start(pb, v0 - _CB)

            return carry

        lax.fori_loop(0, n_w, body, 0)
        blast = lax.rem(n_w - 1, 2)
        vlast = (start_w + n_w - 1) * _CB
        gathers_wait(blast)
        wb_start(blast, vlast)
        wb_wait(1 - blast, vlast - _CB)
        wb_wait(blast, vlast)

    return gather_kernel(table, idx_flat)


def _tc_conv(gs, ws, y3, w2, b2, blk):
    _, nv, nch = y3.shape
    ncols = w2.shape[1]
    nf = ncols // 8

    def body(g0, g1, g2, g3, w0, w1, w2r, w3, y_ref, wc, b_ref, o_ref):
        gr = (g0, g1, g2, g3)
        wr = (w0, w1, w2r, w3)
        acc = jnp.dot(y_ref[0], wc[...], preferred_element_type=jnp.float32)
        for r in range(4):
            acc = acc + jnp.dot(gr[r][...], wr[r][...],
                                preferred_element_type=jnp.float32)
        acc = acc + b_ref[...]
        acc = jnp.maximum(acc, 0.0)
        m = jnp.maximum(acc[:, 0:4 * nf], acc[:, 4 * nf:8 * nf])
        m = jnp.maximum(m[:, 0:2 * nf], m[:, 2 * nf:4 * nf])
        o_ref[0] = jnp.maximum(m[:, 0:nf], m[:, nf:2 * nf])

    g_spec = pl.BlockSpec((blk, ncols), lambda i: (i, 0))
    w_spec = pl.BlockSpec((ncols, ncols), lambda i: (0, 0))
    return pl.pallas_call(
        body,
        grid=(nv // blk,),
        in_specs=[g_spec] * 4 + [w_spec] * 4 + [
            pl.BlockSpec((1, blk, nch), lambda i: (0, i, 0)),
            pl.BlockSpec((nch, ncols), lambda i: (0, 0)),
            pl.BlockSpec((1, ncols), lambda i: (0, 0)),
        ],
        out_specs=pl.BlockSpec((1, blk, nf), lambda i: (0, i, 0)),
        out_shape=jax.ShapeDtypeStruct((1, nv, nf), jnp.float32),
    )(*gs, *ws, y3, w2, b2)


def kernel(y, sync_field, kernel, center_kernel, bias):
    nb, nv, nch = y.shape
    nrings, ndirs, _, nf = kernel.shape

    table = y.reshape(nb * nv, nch)

    dd = jnp.arange(ndirs)
    rot = (dd[:, None] - dd[None, :]) % ndirs
    Krot = kernel[:, rot, :, :]
    W = jnp.transpose(Krot, (0, 1, 3, 2, 4)).reshape(
        nrings, ndirs * nch, ndirs * nf)
    ws = [W[r] for r in range(nrings)]
    w2 = jnp.tile(center_kernel, (1, ndirs))
    b2 = jnp.tile(bias, (ndirs,))[None, :]

    v_idx = sync_field[..., 1]
    table_bf = table.astype(jnp.bfloat16)
    gs = _sc_gather(table_bf, v_idx, nb * nv)
    ws = [w.astype(jnp.bfloat16) for w in ws]
    return _tc_conv(gs, ws, y, w2, b2, blk=2000)


# speedup vs baseline: 1.7250x; 1.7250x over previous
import functools

import jax
import jax.numpy as jnp
from jax import lax
from jax.experimental import pallas as pl
from jax.experimental.pallas import tpu as pltpu
from jax.experimental.pallas import tpu_sc as plsc

_CB = 80
_NRINGS = 4
_NDIRS = 8
_NPOS = _NRINGS * _NDIRS


def _sc_gather(table, idx_flat, nv):
    nch = table.shape[1]
    gw = _NDIRS * nch
    info = plsc.get_sparse_core_info()
    nw = info.num_cores * info.num_subcores
    n_chunks = nv // _CB
    base_per_w = n_chunks // nw
    extra = n_chunks - base_per_w * nw
    mesh = plsc.VectorSubcoreMesh(core_axis_name="c", subcore_axis_name="s")

    @functools.partial(
        pl.kernel,
        mesh=mesh,
        compiler_params=pltpu.CompilerParams(
            use_tc_tiling_on_sc=False, needs_layout_passes=False),
        out_type=[jax.ShapeDtypeStruct((nv, gw), jnp.float32)
                  for _ in range(_NRINGS)],
        scratch_types=[
            pltpu.VMEM((_CB, _NRINGS, _NDIRS), jnp.int32),
            pltpu.VMEM((2, _NPOS, _CB), jnp.int32),
            pltpu.VMEM((2, _NRINGS, _NDIRS, _CB, nch), jnp.float32),
            pltpu.SemaphoreType.DMA((2, _NRINGS)),
            pltpu.SemaphoreType.DMA((2, _NRINGS)),
        ],
    )
    def gather_kernel(table_hbm, idx_hbm, o0, o1, o2, o3,
                      idxv, idxp, rows_v, gsem, wsem):
        outs = [o0, o1, o2, o3]
        wid = lax.axis_index("s") * info.num_cores + lax.axis_index("c")
        start_w = wid * base_per_w + lax.min(wid, extra)
        n_w = base_per_w + jnp.where(wid < extra, 1, 0)
        lane = lax.iota(jnp.int32, 16)
        ones = jnp.full((16,), 1, jnp.int32)

        def gathers_start(b, chunk):
            for r in range(_NRINGS):
                for dd in range(_NDIRS):
                    pltpu.async_copy(
                        table_hbm.at[idxp.at[b, r * _NDIRS + dd]],
                        rows_v.at[b, r, dd], gsem.at[b, r])

        def gathers_wait(b):
            for r in range(_NRINGS):
                for dd in range(_NDIRS):
                    pltpu.make_async_copy(
                        table_hbm.at[idxp.at[b, r * _NDIRS + dd]],
                        rows_v.at[b, r, dd], gsem.at[b, r]).wait()

        def wb_start(b, v0):
            for r in range(_NRINGS):
                for dd in range(_NDIRS):
                    pltpu.async_copy(
                        rows_v.at[b, r, dd],
                        outs[r].at[pl.ds(v0, _CB), pl.ds(dd * nch, nch)],
                        wsem.at[b, r])

        def wb_wait(b, v0):
            for r in range(_NRINGS):
                for dd in range(_NDIRS):
                    pltpu.make_async_copy(
                        rows_v.at[b, r, dd],
                        outs[r].at[pl.ds(v0, _CB), pl.ds(dd * nch, nch)],
                        wsem.at[b, r]).wait()

        def body(j, carry):
            b = lax.rem(j, 2)
            pb = 1 - b
            chunk = start_w + j
            v0 = chunk * _CB
            pltpu.sync_copy(idx_hbm.at[0, pl.ds(v0, _CB)], idxv)
            for r in range(_NRINGS):
                for dd in range(_NDIRS):
                    p = r * _NDIRS + dd
                    for k in range(_CB // 16):
                        vals = plsc.load_gather(
                            idxv, [lane + 16 * k, ones * r, ones * dd])
                        idxp[b, p, pl.ds(16 * k, 16)] = vals

            @pl.when(j > 1)
            def _():
                wb_wait(b, v0 - 2 * _CB)

            gathers_start(b, chunk)

            @pl.when(j > 0)
            def _():
                gathers_wait(pb)
                wb_start(pb, v0 - _CB)

            return carry

        lax.fori_loop(0, n_w, body, 0)
        blast = lax.rem(n_w - 1, 2)
        vlast = (start_w + n_w - 1) * _CB
        gathers_wait(blast)
        wb_start(blast, vlast)
        wb_wait(1 - blast, vlast - _CB)
        wb_wait(blast, vlast)

    return gather_kernel(table, idx_flat)


def _tc_conv(gs, ws, y3, w2, b2, blk):
    _, nv, nch = y3.shape
    ncols = w2.shape[1]
    nf = ncols // 8

    def body(g0, g1, g2, g3, w0, w1, w2r, w3, y_ref, wc, b_ref, o_ref):
        gr = (g0, g1, g2, g3)
        wr = (w0, w1, w2r, w3)
        acc = jnp.dot(y_ref[0], wc[...], preferred_element_type=jnp.float32)
        for r in range(4):
            acc = acc + jnp.dot(gr[r][...], wr[r][...],
                                preferred_element_type=jnp.float32)
        acc = acc + b_ref[...]
        acc = jnp.maximum(acc, 0.0)
        m = jnp.maximum(acc[:, 0:4 * nf], acc[:, 4 * nf:8 * nf])
        m = jnp.maximum(m[:, 0:2 * nf], m[:, 2 * nf:4 * nf])
        o_ref[0] = jnp.maximum(m[:, 0:nf], m[:, nf:2 * nf])

    g_spec = pl.BlockSpec((blk, ncols), lambda i: (i, 0))
    w_spec = pl.BlockSpec((ncols, ncols), lambda i: (0, 0))
    return pl.pallas_call(
        body,
        grid=(nv // blk,),
        in_specs=[g_spec] * 4 + [w_spec] * 4 + [
            pl.BlockSpec((1, blk, nch), lambda i: (0, i, 0)),
            pl.BlockSpec((nch, ncols), lambda i: (0, 0)),
            pl.BlockSpec((1, ncols), lambda i: (0, 0)),
        ],
        out_specs=pl.BlockSpec((1, blk, nf), lambda i: (0, i, 0)),
        out_shape=jax.ShapeDtypeStruct((1, nv, nf), jnp.float32),
    )(*gs, *ws, y3, w2, b2)


def kernel(y, sync_field, kernel, center_kernel, bias):
    nb, nv, nch = y.shape
    nrings, ndirs, _, nf = kernel.shape

    table = y.reshape(nb * nv, nch)

    dd = jnp.arange(ndirs)
    rot = (dd[:, None] - dd[None, :]) % ndirs
    Krot = kernel[:, rot, :, :]
    W = jnp.transpose(Krot, (0, 1, 3, 2, 4)).reshape(
        nrings, ndirs * nch, ndirs * nf)
    ws = [W[r] for r in range(nrings)]
    w2 = jnp.tile(center_kernel, (1, ndirs))
    b2 = jnp.tile(bias, (ndirs,))[None, :]

    v_idx = sync_field[..., 1]
    gs = _sc_gather(table, v_idx, nb * nv)
    return _tc_conv(gs, ws, y, w2, b2, blk=2000)
